# Initial kernel scaffold; baseline (speedup 1.0000x reference)
import jax
import jax.numpy as jnp
from jax.experimental import pallas as pl


def kernel(x, adjs, epoch, syn_emb, pos_emb, sen_emb, lem_emb, ln_gamma, ln_beta, W, b):
    raise NotImplementedError("write your pallas kernel here")


# speedup vs baseline: 7.0987x; 7.0987x over previous
import jax
import jax.numpy as jnp
from jax import lax
from jax.experimental import pallas as pl
from jax.experimental.pallas import tpu as pltpu
from jax.experimental.pallas import tpu_sc as plsc

N = 10000
D = 128
E = 320000
NC = 2
NS = 16
NW = NC * NS
CH = 128
NODE_CH = 3
N_PAD = NW * NODE_CH * CH
EC = 80
EPH = 2
EC_PH = EC // EPH
E_PAD = NW * EC * CH
TRASH = N
DEG_STRIPE = N_PAD // NS
ACC_STRIPE = 640
ACC_ROWS = NS * ACC_STRIPE
LN_EPS = 1e-12

_mesh = plsc.VectorSubcoreMesh(core_axis_name="c", subcore_axis_name="s")


def _k1_body(syn_emb, pos_emb, sen_emb, lem_emb, idx_all, dst_i,
             ones_hbm, zdeg_hbm,
             emb_out, deg_out,
             idx_v, dstv, ones_v, deg_sp):
    cid = lax.axis_index("c")
    sid = lax.axis_index("s")
    wid = cid * NS + sid

    pltpu.sync_copy(zdeg_hbm, deg_sp.at[pl.ds(sid * DEG_STRIPE, DEG_STRIPE)])
    pltpu.sync_copy(ones_hbm, ones_v)
    pltpu.sync_copy(dst_i.at[wid], dstv)
    plsc.subcore_barrier()

    @pl.loop(0, EC)
    def _(ch):
        pltpu.sync_copy(ones_v, deg_sp.at[dstv.at[ch]], add=True)

    pltpu.sync_copy(idx_all.at[wid], idx_v)
    tables = (syn_emb, pos_emb, sen_emb, lem_emb)

    def _emb(rows):
        for t in range(4):
            for c in range(NODE_CH):
                pltpu.sync_copy(tables[t].at[idx_v.at[t * NODE_CH + c]], rows)
                pltpu.sync_copy(
                    rows,
                    emb_out.at[t, pl.ds(wid * NODE_CH * CH + c * CH, CH)])

    pl.run_scoped(_emb, pltpu.VMEM((CH, D), jnp.float32))

    plsc.subcore_barrier()
    pltpu.sync_copy(deg_sp.at[pl.ds(sid * DEG_STRIPE, DEG_STRIPE)],
                    deg_out.at[cid, pl.ds(sid * DEG_STRIPE, DEG_STRIPE)])


def _k3_body(hw2, src_i, dst_i,
             parts_out,
             srcv, dstv, acc_sp):
    cid = lax.axis_index("c")
    sid = lax.axis_index("s")
    wid = cid * NS + sid

    def _main(rows):
        @pl.loop(0, CH)
        def _(i):
            for j in range(8):
                rows[i, pl.ds(j * 16, 16)] = jnp.zeros((16,), jnp.float32)

        @pl.when(cid == 0)
        def _():
            pltpu.sync_copy(hw2.at[pl.ds(sid * ACC_STRIPE, ACC_STRIPE)],
                            acc_sp.at[pl.ds(sid * ACC_STRIPE, ACC_STRIPE)])

        @pl.when(cid != 0)
        def _():
            for k in range(ACC_STRIPE // CH):
                pltpu.sync_copy(
                    rows, acc_sp.at[pl.ds(sid * ACC_STRIPE + k * CH, CH)])

        plsc.subcore_barrier()

        for ph in range(EPH):
            pltpu.sync_copy(src_i.at[wid, pl.ds(ph * EC_PH, EC_PH)], srcv)
            pltpu.sync_copy(dst_i.at[wid, pl.ds(ph * EC_PH, EC_PH)], dstv)

            @pl.loop(0, EC_PH)
            def _(ch):
                pltpu.sync_copy(hw2.at[srcv.at[ch]], rows)
                pltpu.sync_copy(rows, acc_sp.at[dstv.at[ch]], add=True)

        plsc.subcore_barrier()
        pltpu.sync_copy(acc_sp.at[pl.ds(sid * ACC_STRIPE, ACC_STRIPE)],
                        parts_out.at[cid, pl.ds(sid * ACC_STRIPE, ACC_STRIPE)])

    pl.run_scoped(_main, pltpu.VMEM((CH, D), jnp.float32))


_k1 = pl.kernel(
    _k1_body,
    out_type=[jax.ShapeDtypeStruct((4, N_PAD, D), jnp.float32),
              jax.ShapeDtypeStruct((NC, N_PAD, 16), jnp.float32)],
    mesh=_mesh,
    scratch_types=[
        pltpu.VMEM((4 * NODE_CH, CH), jnp.int32),
        pltpu.VMEM((EC, CH), jnp.int32),
        pltpu.VMEM((CH, 16), jnp.float32),
        pltpu.VMEM_SHARED((N_PAD, 16), jnp.float32),
    ],
)

_k3 = pl.kernel(
    _k3_body,
    out_type=jax.ShapeDtypeStruct((NC, ACC_ROWS, D), jnp.float32),
    mesh=_mesh,
    scratch_types=[
        pltpu.VMEM((EC_PH, CH), jnp.int32),
        pltpu.VMEM((EC_PH, CH), jnp.int32),
        pltpu.VMEM_SHARED((ACC_ROWS, D), jnp.float32),
    ],
)


_RB = 512


def _k2_body(emb_ref, dc_ref, g_ref, bt_ref, w_ref, hw2_ref, dinv_ref):
    h = ((emb_ref[0] + emb_ref[3]) + emb_ref[1]) + emb_ref[2]
    mu = jnp.mean(h, axis=-1, keepdims=True)
    xc = h - mu
    var = jnp.mean(xc * xc, axis=-1, keepdims=True)
    hn = xc * lax.rsqrt(var + LN_EPS) * g_ref[0] + bt_ref[0]
    deg = (dc_ref[0] + dc_ref[1])[:, 0:1] + 1.0
    dinv = lax.rsqrt(deg)
    hw = jnp.dot(hn, w_ref[...], preferred_element_type=jnp.float32,
                 precision=lax.Precision.HIGHEST)
    hw2_ref[...] = hw * dinv
    dinv_ref[...] = dinv


_RB4 = 400


def _k4_body(parts_ref, dinv_ref, b_ref, out_ref):
    s = parts_ref[0] + parts_ref[1]
    out_ref[...] = jnp.maximum(s * dinv_ref[...] + b_ref[0], 0.0)


def kernel(x, adjs, epoch, syn_emb, pos_emb, sen_emb, lem_emb,
           ln_gamma, ln_beta, W, b):
    x = x.astype(jnp.int32)
    adjs = adjs.astype(jnp.int32)
    idx_cols = [jnp.pad(x[:, k], (0, N_PAD - N)) for k in range(4)]
    idx_all = jnp.stack(idx_cols, axis=0).reshape(4, NW, NODE_CH, CH)
    idx_all = jnp.transpose(idx_all, (1, 0, 2, 3)).reshape(NW, 4 * NODE_CH, CH)
    src_p = jnp.pad(adjs[0], (0, E_PAD - E)).reshape(NW, EC, CH)
    dst_p = jnp.pad(adjs[1], (0, E_PAD - E),
                    constant_values=TRASH).reshape(NW, EC, CH)
    ones_deg = jnp.ones((CH, 16), jnp.float32)
    zeros_deg = jnp.zeros((DEG_STRIPE, 16), jnp.float32)
    g2 = ln_gamma.reshape(1, D)
    bt2 = ln_beta.reshape(1, D)
    b2 = b.reshape(1, D)

    emb, degcnt = _k1(syn_emb, pos_emb, sen_emb, lem_emb, idx_all, dst_p,
                      ones_deg, zeros_deg)

    hw2, dinv = pl.pallas_call(
        _k2_body,
        grid=(N_PAD // _RB,),
        in_specs=[
            pl.BlockSpec((4, _RB, D), lambda i: (0, i, 0)),
            pl.BlockSpec((NC, _RB, 16), lambda i: (0, i, 0)),
            pl.BlockSpec((1, D), lambda i: (0, 0)),
            pl.BlockSpec((1, D), lambda i: (0, 0)),
            pl.BlockSpec((D, D), lambda i: (0, 0)),
        ],
        out_specs=[pl.BlockSpec((_RB, D), lambda i: (i, 0)),
                   pl.BlockSpec((_RB, 1), lambda i: (i, 0))],
        out_shape=[jax.ShapeDtypeStruct((N_PAD, D), jnp.float32),
                   jax.ShapeDtypeStruct((N_PAD, 1), jnp.float32)],
    )(emb, degcnt, g2, bt2, W)

    parts = _k3(hw2, src_p, dst_p)

    out = pl.pallas_call(
        _k4_body,
        grid=(N // _RB4,),
        in_specs=[
            pl.BlockSpec((NC, _RB4, D), lambda i: (0, i, 0)),
            pl.BlockSpec((_RB4, 1), lambda i: (i, 0)),
            pl.BlockSpec((1, D), lambda i: (0, 0)),
        ],
        out_specs=pl.BlockSpec((_RB4, D), lambda i: (i, 0)),
        out_shape=jax.ShapeDtypeStruct((N, D), jnp.float32),
    )(parts, dinv, b2)
    return out


# speedup vs baseline: 7.9316x; 1.1173x over previous
import dataclasses

import jax
import jax.numpy as jnp
from jax import lax
from jax.experimental import pallas as pl
from jax.experimental.pallas import tpu as pltpu
from jax.experimental.pallas import tpu_sc as plsc

N = 10000
D = 128
E = 320000
NC = 2
NS = 16
NW = NC * NS
CH = 128
NODE_CH = 3
N_PAD = NW * NODE_CH * CH
EC = 80
EPH = 2
EC_PH = EC // EPH
E_PAD = NW * EC * CH
TRASH = N
DEG_STRIPE = N_PAD // NS
ACC_STRIPE = 640
ACC_ROWS = NS * ACC_STRIPE
LN_EPS = 1e-12

_mesh = plsc.VectorSubcoreMesh(core_axis_name="c", subcore_axis_name="s")

_cp = pltpu.CompilerParams()
if "needs_layout_passes" in pltpu.CompilerParams.__dataclass_fields__:
    _cp = dataclasses.replace(_cp, needs_layout_passes=False)


def _k1_body(syn_emb, pos_emb, sen_emb, lem_emb, idx_all, dst_i,
             zdeg_hbm, iota_hbm,
             emb_out, deg_out,
             idx_v, dstv, hist_v, iota_v, deg_sp):
    cid = lax.axis_index("c")
    sid = lax.axis_index("s")
    wid = cid * NS + sid

    @pl.when(sid < 12)
    def _():
        pltpu.sync_copy(zdeg_hbm, deg_sp.at[pl.ds(sid * 8, 8)])

    pltpu.sync_copy(dst_i.at[wid], dstv)
    pltpu.sync_copy(iota_hbm, iota_v)

    @pl.loop(0, N_PAD // CH)
    def _(i):
        for j in range(8):
            hist_v[i, pl.ds(j * 16, 16)] = jnp.zeros((16,), jnp.float32)

    ones16 = jnp.full((16,), 1.0, jnp.float32)

    @pl.loop(0, EC)
    def _(ch):
        for j in range(8):
            idx = dstv[ch, pl.ds(j * 16, 16)]
            r = lax.shift_right_logical(idx, 7)
            c = lax.bitwise_and(idx, 127)
            plsc.addupdate_scatter(hist_v, [r, c], ones16)

    plsc.subcore_barrier()
    pltpu.sync_copy(hist_v, deg_sp.at[iota_v.at[0]], add=True)

    pltpu.sync_copy(idx_all.at[wid], idx_v)
    tables = (syn_emb, pos_emb, sen_emb, lem_emb)

    def _emb(rows):
        for t in range(4):
            for c in range(NODE_CH):
                pltpu.sync_copy(tables[t].at[idx_v.at[t * NODE_CH + c]], rows)
                pltpu.sync_copy(
                    rows,
                    emb_out.at[t, pl.ds(wid * NODE_CH * CH + c * CH, CH)])

    pl.run_scoped(_emb, pltpu.VMEM((CH, D), jnp.float32))

    plsc.subcore_barrier()

    @pl.when(sid < 12)
    def _():
        pltpu.sync_copy(deg_sp.at[pl.ds(sid * 8, 8)],
                        deg_out.at[cid, pl.ds(sid * 8, 8)])


def _k3_body(hw2, src_i, dst_i,
             parts_out,
             srcv, dstv, acc_sp):
    cid = lax.axis_index("c")
    sid = lax.axis_index("s")
    wid = cid * NS + sid

    def _main(rows):
        @pl.loop(0, CH)
        def _(i):
            for j in range(8):
                rows[i, pl.ds(j * 16, 16)] = jnp.zeros((16,), jnp.float32)

        @pl.when(cid == 0)
        def _():
            pltpu.sync_copy(hw2.at[pl.ds(sid * ACC_STRIPE, ACC_STRIPE)],
                            acc_sp.at[pl.ds(sid * ACC_STRIPE, ACC_STRIPE)])

        @pl.when(cid != 0)
        def _():
            for k in range(ACC_STRIPE // CH):
                pltpu.sync_copy(
                    rows, acc_sp.at[pl.ds(sid * ACC_STRIPE + k * CH, CH)])

        plsc.subcore_barrier()

        for ph in range(EPH):
            pltpu.sync_copy(src_i.at[wid, pl.ds(ph * EC_PH, EC_PH)], srcv)
            pltpu.sync_copy(dst_i.at[wid, pl.ds(ph * EC_PH, EC_PH)], dstv)

            @pl.loop(0, EC_PH)
            def _(ch):
                pltpu.sync_copy(hw2.at[srcv.at[ch]], rows)
                pltpu.sync_copy(rows, acc_sp.at[dstv.at[ch]], add=True)

        plsc.subcore_barrier()
        pltpu.sync_copy(acc_sp.at[pl.ds(sid * ACC_STRIPE, ACC_STRIPE)],
                        parts_out.at[cid, pl.ds(sid * ACC_STRIPE, ACC_STRIPE)])

    pl.run_scoped(_main, pltpu.VMEM((CH, D), jnp.float32))


_k1 = pl.kernel(
    _k1_body,
    out_type=[jax.ShapeDtypeStruct((4, N_PAD, D), jnp.float32),
              jax.ShapeDtypeStruct((NC, N_PAD // CH, CH), jnp.float32)],
    mesh=_mesh,
    scratch_types=[
        pltpu.VMEM((4 * NODE_CH, CH), jnp.int32),
        pltpu.VMEM((EC, CH), jnp.int32),
        pltpu.VMEM((N_PAD // CH, CH), jnp.float32),
        pltpu.VMEM((1, N_PAD // CH), jnp.int32),
        pltpu.VMEM_SHARED((N_PAD // CH, CH), jnp.float32),
    ],
    compiler_params=_cp,
)

_k3 = pl.kernel(
    _k3_body,
    out_type=jax.ShapeDtypeStruct((NC, ACC_ROWS, D), jnp.float32),
    mesh=_mesh,
    scratch_types=[
        pltpu.VMEM((EC_PH, CH), jnp.int32),
        pltpu.VMEM((EC_PH, CH), jnp.int32),
        pltpu.VMEM_SHARED((ACC_ROWS, D), jnp.float32),
    ],
)


_RB = 512


def _k2_body(emb_ref, dc_ref, g_ref, bt_ref, w_ref, hw2_ref, dinv_ref):
    h = ((emb_ref[0] + emb_ref[3]) + emb_ref[1]) + emb_ref[2]
    mu = jnp.mean(h, axis=-1, keepdims=True)
    xc = h - mu
    var = jnp.mean(xc * xc, axis=-1, keepdims=True)
    hn = xc * lax.rsqrt(var + LN_EPS) * g_ref[0] + bt_ref[0]
    deg = (dc_ref[0] + dc_ref[1]) + 1.0
    dinv = lax.rsqrt(deg)
    hw = jnp.dot(hn, w_ref[...], preferred_element_type=jnp.float32,
                 precision=lax.Precision.HIGHEST)
    hw2_ref[...] = hw * dinv
    dinv_ref[...] = dinv


_RB4 = 400


def _k4_body(parts_ref, dinv_ref, b_ref, out_ref):
    s = parts_ref[0] + parts_ref[1]
    out_ref[...] = jnp.maximum(s * dinv_ref[...] + b_ref[0], 0.0)


def kernel(x, adjs, epoch, syn_emb, pos_emb, sen_emb, lem_emb,
           ln_gamma, ln_beta, W, b):
    x = x.astype(jnp.int32)
    adjs = adjs.astype(jnp.int32)
    idx_cols = [jnp.pad(x[:, k], (0, N_PAD - N)) for k in range(4)]
    idx_all = jnp.stack(idx_cols, axis=0).reshape(4, NW, NODE_CH, CH)
    idx_all = jnp.transpose(idx_all, (1, 0, 2, 3)).reshape(NW, 4 * NODE_CH, CH)
    src_p = jnp.pad(adjs[0], (0, E_PAD - E)).reshape(NW, EC, CH)
    dst_p = jnp.pad(adjs[1], (0, E_PAD - E),
                    constant_values=TRASH).reshape(NW, EC, CH)
    zeros_deg = jnp.zeros((8, CH), jnp.float32)
    iota96 = jnp.arange(N_PAD // CH, dtype=jnp.int32).reshape(1, N_PAD // CH)
    g2 = ln_gamma.reshape(1, D)
    bt2 = ln_beta.reshape(1, D)
    b2 = b.reshape(1, D)

    emb, degcnt = _k1(syn_emb, pos_emb, sen_emb, lem_emb, idx_all, dst_p,
                      zeros_deg, iota96)
    degcnt = degcnt.reshape(NC, N_PAD, 1)

    hw2, dinv = pl.pallas_call(
        _k2_body,
        grid=(N_PAD // _RB,),
        in_specs=[
            pl.BlockSpec((4, _RB, D), lambda i: (0, i, 0)),
            pl.BlockSpec((NC, _RB, 1), lambda i: (0, i, 0)),
            pl.BlockSpec((1, D), lambda i: (0, 0)),
            pl.BlockSpec((1, D), lambda i: (0, 0)),
            pl.BlockSpec((D, D), lambda i: (0, 0)),
        ],
        out_specs=[pl.BlockSpec((_RB, D), lambda i: (i, 0)),
                   pl.BlockSpec((_RB, 1), lambda i: (i, 0))],
        out_shape=[jax.ShapeDtypeStruct((N_PAD, D), jnp.float32),
                   jax.ShapeDtypeStruct((N_PAD, 1), jnp.float32)],
    )(emb, degcnt, g2, bt2, W)

    parts = _k3(hw2, src_p, dst_p)

    out = pl.pallas_call(
        _k4_body,
        grid=(N // _RB4,),
        in_specs=[
            pl.BlockSpec((NC, _RB4, D), lambda i: (0, i, 0)),
            pl.BlockSpec((_RB4, 1), lambda i: (i, 0)),
            pl.BlockSpec((1, D), lambda i: (0, 0)),
        ],
        out_specs=pl.BlockSpec((_RB4, D), lambda i: (i, 0)),
        out_shape=jax.ShapeDtypeStruct((N, D), jnp.float32),
    )(parts, dinv, b2)
    return out
